# speedup vs baseline: 9.2268x; 9.2268x over previous
import functools

import jax
import jax.numpy as jnp
from jax import lax
from jax.experimental import pallas as pl
from jax.experimental.pallas import tpu as pltpu
from jax.experimental.pallas import tpu_sc as plsc

D = 128
CPR = 128
NBUF = 4


@functools.partial(jax.jit, static_argnames=("nw", "chunks"))
def _gather_rows(pe, idx2d, *, nw, chunks):
    rows_per_w = chunks * CPR
    mesh = plsc.VectorSubcoreMesh(core_axis_name="c", subcore_axis_name="s")
    nc = mesh.num_cores

    @functools.partial(
        pl.kernel,
        out_type=jax.ShapeDtypeStruct((nw * rows_per_w, D), jnp.float32),
        mesh=mesh,
        scratch_types=[
            pltpu.VMEM((chunks, CPR), jnp.int32),
            pltpu.VMEM((NBUF, CPR, D), jnp.float32),
            pltpu.SemaphoreType.DMA((NBUF,)),
            pltpu.SemaphoreType.DMA((NBUF,)),
        ],
    )
    def k(pe_hbm, idx_hbm, out_hbm, idx_v, rows, gsem, ssem):
        wid = lax.axis_index("s") * nc + lax.axis_index("c")
        row0 = wid * rows_per_w
        pltpu.sync_copy(idx_hbm.at[pl.ds(wid * chunks, chunks)], idx_v)

        for b in range(NBUF):
            pltpu.async_copy(pe_hbm.at[idx_v.at[b]], rows.at[b], gsem.at[b])

        @pl.loop(0, chunks - NBUF, step=NBUF)
        def _(j0):
            for b in range(NBUF):
                j = j0 + b
                dst = out_hbm.at[pl.ds(row0 + j * CPR, CPR)]
                pltpu.make_async_copy(
                    pe_hbm.at[idx_v.at[j]], rows.at[b], gsem.at[b]
                ).wait()
                pltpu.async_copy(rows.at[b], dst, ssem.at[b])
                pltpu.make_async_copy(rows.at[b], dst, ssem.at[b]).wait()
                pltpu.async_copy(
                    pe_hbm.at[idx_v.at[j + NBUF]], rows.at[b], gsem.at[b]
                )

        for b in range(NBUF):
            j = chunks - NBUF + b
            dst = out_hbm.at[pl.ds(row0 + j * CPR, CPR)]
            pltpu.make_async_copy(
                pe_hbm.at[idx_v.at[j]], rows.at[b], gsem.at[b]
            ).wait()
            pltpu.async_copy(rows.at[b], dst, ssem.at[b])
            pltpu.make_async_copy(rows.at[b], dst, ssem.at[b]).wait()

    return k(pe, idx2d)


def kernel(timesteps, pe):
    bsz, hist = timesteps.shape
    total = bsz * hist
    nw = 32
    chunks = total // (nw * CPR)
    assert chunks * nw * CPR == total
    idx2d = timesteps.reshape(nw * chunks, CPR)
    out = _gather_rows(pe, idx2d, nw=nw, chunks=chunks)
    return out.reshape(bsz, hist, pe.shape[1])
